# baseline (speedup 1.0000x reference)
import jax
import jax.numpy as jnp
from jax import lax
from jax.experimental import pallas as pl
from jax.experimental.pallas import tpu as pltpu
from jax.experimental.pallas import tpu_sc as plsc

NC, NS = 2, 16
NW = NC * NS
L = 16

B = 16384
D = 32
BPW = B // NW
SPW = 128
NST = BPW // SPW


SEMQ = 8


def _gmf_body(users_hbm, items_hbm, ut_hbm, it_hbm, w_hbm, b_hbm, out_hbm,
              uidx_v, iidx_v,
              u_rows0, u_rows1, i_rows0, i_rows1,
              w_v, b_v, out_v, *sems):
    wid = lax.axis_index("s") * NC + lax.axis_index("c")
    base = wid * BPW

    pltpu.sync_copy(users_hbm.at[pl.ds(base, BPW)], uidx_v)
    pltpu.sync_copy(items_hbm.at[pl.ds(base, BPW)], iidx_v)
    pltpu.sync_copy(w_hbm, w_v)
    pltpu.sync_copy(b_hbm, b_v)

    u_bufs = [u_rows0, u_rows1]
    i_bufs = [i_rows0, i_rows1]

    def fire(s):
        p = s % 2
        qs = sems[p * SEMQ:(p + 1) * SEMQ]

        @plsc.parallel_loop(0, SPW, step=L)
        def fetch_body(j):
            uvec = uidx_v[pl.ds(s * SPW + j, L)]
            ivec = iidx_v[pl.ds(s * SPW + j, L)]
            for k in range(L):
                q = qs[k % SEMQ]
                pltpu.async_copy(
                    ut_hbm.at[uvec[k]], u_bufs[p].at[j + k], q)
                pltpu.async_copy(
                    it_hbm.at[ivec[k]], i_bufs[p].at[j + k], q)

    def drain(s):
        p = s % 2
        qs = sems[p * SEMQ:(p + 1) * SEMQ]
        rpq = SPW // SEMQ
        dummy = ut_hbm.at[pl.ds(0, rpq)]
        for q in qs:
            pltpu.make_async_copy(dummy, u_bufs[p].at[pl.ds(0, rpq)], q).wait()
            pltpu.make_async_copy(dummy, i_bufs[p].at[pl.ds(0, rpq)], q).wait()

    b_vec = b_v[...]
    w_lo = w_v[pl.ds(0, L)]
    w_hi = w_v[pl.ds(L, L)]
    w_s = [w_lo[d] for d in range(L)] + [w_hi[d] for d in range(L)]
    lane = lax.iota(jnp.int32, L)
    cols = [jnp.full((L,), d, jnp.int32) for d in range(D)]

    fire(0)
    for s in range(NST):
        if s + 1 < NST:
            fire(s + 1)
        drain(s)
        p = s % 2
        ub = u_bufs[p]
        ib = i_bufs[p]

        def group_body(g, carry, s=s, ub=ub, ib=ib):
            slots = g * L + lane
            acc = jnp.zeros((L,), jnp.float32)
            for d in range(D):
                ug = plsc.load_gather(ub, [slots, cols[d]])
                ig = plsc.load_gather(ib, [slots, cols[d]])
                acc = acc + ug * ig * w_s[d]
            logits = acc + b_vec
            preds = 1.0 / (1.0 + jnp.exp(-logits))
            out_v[pl.ds(s * SPW + g * L, L)] = preds
            return carry

        lax.fori_loop(0, SPW // L, group_body, 0)

    pltpu.sync_copy(out_v, out_hbm.at[pl.ds(base, BPW)])


@jax.jit
def kernel(users, items, user_table, item_table, W, b):
    mesh = plsc.VectorSubcoreMesh(
        core_axis_name="c", subcore_axis_name="s",
        num_cores=NC, num_subcores=NS)
    run = pl.kernel(
        _gmf_body,
        out_type=jax.ShapeDtypeStruct((B,), jnp.float32),
        mesh=mesh,
        scratch_types=[
            pltpu.VMEM((BPW,), jnp.int32),
            pltpu.VMEM((BPW,), jnp.int32),
            pltpu.VMEM((SPW, D), jnp.float32),
            pltpu.VMEM((SPW, D), jnp.float32),
            pltpu.VMEM((SPW, D), jnp.float32),
            pltpu.VMEM((SPW, D), jnp.float32),
            pltpu.VMEM((D,), jnp.float32),
            pltpu.VMEM((L,), jnp.float32),
            pltpu.VMEM((BPW,), jnp.float32),
        ] + [pltpu.SemaphoreType.DMA] * (2 * SEMQ),
        compiler_params=pltpu.CompilerParams(needs_layout_passes=False),
    )
    w32 = W.reshape(D).astype(jnp.float32)
    b16 = jnp.broadcast_to(b.astype(jnp.float32), (L,))
    out = run(users.astype(jnp.int32), items.astype(jnp.int32),
              user_table, item_table, w32, b16)
    return out.reshape(B, 1)
